# baseline (speedup 1.0000x reference)
import functools

import jax
import jax.numpy as jnp
from jax import lax
from jax.experimental import pallas as pl
from jax.experimental.pallas import tpu as pltpu
from jax.experimental.pallas import tpu_sc as plsc

D = 64
WINDOW = 8
LANES = 16
CHUNK = 512


def kernel(input, table, b_embed):
    B, N = input.shape
    BN = B * N
    idx_flat = input.reshape(BN).astype(jnp.int32)

    info = plsc.get_sparse_core_info()
    num_workers = info.num_cores * info.num_subcores
    per_w = BN // num_workers
    assert per_w * num_workers == BN and per_w % CHUNK == 0
    n_chunks = per_w // CHUNK

    @functools.partial(
        pl.kernel,
        out_type=jax.ShapeDtypeStruct((BN, D), jnp.float32),
        mesh=plsc.VectorSubcoreMesh(core_axis_name="c", subcore_axis_name="s"),
        compiler_params=pltpu.CompilerParams(use_tc_tiling_on_sc=False),
        scratch_types=[
            pltpu.VMEM((CHUNK,), jnp.int32),
            pltpu.VMEM((CHUNK, D), jnp.float32),
            pltpu.VMEM((D,), jnp.float32),
            pltpu.SemaphoreType.DMA,
        ],
    )
    def body(idx_hbm, table_hbm, bias_hbm, out_hbm, idx_v, rows_v, b_v, sem):
        wid = lax.axis_index("s") * info.num_cores + lax.axis_index("c")
        base = wid * per_w
        pltpu.sync_copy(bias_hbm, b_v)

        def chunk_body(c, _):
            off = base + c * CHUNK
            pltpu.sync_copy(idx_hbm.at[pl.ds(off, CHUNK)], idx_v)
            pltpu.async_copy(table_hbm.at[idx_v], rows_v, sem).wait()

            def beacon_row(r, _):
                row = r * WINDOW
                for j in range(D // LANES):
                    sl = pl.ds(j * LANES, LANES)
                    rows_v[row, sl] = rows_v[row, sl] + b_v[sl]
                return 0

            lax.fori_loop(0, CHUNK // WINDOW, beacon_row, 0)
            pltpu.sync_copy(rows_v, out_hbm.at[pl.ds(off, CHUNK)])
            return 0

        lax.fori_loop(0, n_chunks, chunk_body, 0)

    out = body(idx_flat, table, b_embed)
    return out.reshape(B, N, D)


# speedup vs baseline: 1.0940x; 1.0940x over previous
import functools

import jax
import jax.numpy as jnp
from jax import lax
from jax.experimental import pallas as pl
from jax.experimental.pallas import tpu as pltpu
from jax.experimental.pallas import tpu_sc as plsc

D = 64
WINDOW = 8
LANES = 16
CHUNK = 256
NBUF = 4


def kernel(input, table, b_embed):
    B, N = input.shape
    BN = B * N
    idx_flat = input.reshape(BN).astype(jnp.int32)

    info = plsc.get_sparse_core_info()
    num_workers = info.num_cores * info.num_subcores
    per_w = BN // num_workers
    assert per_w * num_workers == BN and per_w % (CHUNK * NBUF) == 0
    n_chunks = per_w // CHUNK
    n_outer = n_chunks // NBUF

    @functools.partial(
        pl.kernel,
        out_type=jax.ShapeDtypeStruct((BN, D), jnp.float32),
        mesh=plsc.VectorSubcoreMesh(core_axis_name="c", subcore_axis_name="s"),
        compiler_params=pltpu.CompilerParams(use_tc_tiling_on_sc=False),
        scratch_types=[
            pltpu.VMEM((per_w,), jnp.int32),
            pltpu.VMEM((NBUF, CHUNK, D), jnp.float32),
            pltpu.VMEM((D,), jnp.float32),
        ]
        + [pltpu.SemaphoreType.DMA] * (2 * NBUF),
    )
    def body(idx_hbm, table_hbm, bias_hbm, out_hbm, idx_all, rows, b_v, *sems):
        gsem = sems[:NBUF]
        ssem = sems[NBUF:]
        wid = lax.axis_index("s") * info.num_cores + lax.axis_index("c")
        base = wid * per_w
        pltpu.sync_copy(bias_hbm, b_v)
        pltpu.sync_copy(idx_hbm.at[pl.ds(base, per_w)], idx_all)

        def gather_start(c, p):
            src = table_hbm.at[idx_all.at[pl.ds(c * CHUNK, CHUNK)]]
            pltpu.async_copy(src, rows.at[p], gsem[p])

        def gather_wait(p):
            src = table_hbm.at[idx_all.at[pl.ds(0, CHUNK)]]
            pltpu.make_async_copy(src, rows.at[p], gsem[p]).wait()

        def store_start(c, p):
            dst = out_hbm.at[pl.ds(base + c * CHUNK, CHUNK)]
            pltpu.async_copy(rows.at[p], dst, ssem[p])

        def store_wait(p):
            dst = out_hbm.at[pl.ds(base, CHUNK)]
            pltpu.make_async_copy(rows.at[p], dst, ssem[p]).wait()

        def add_bias(p):
            def beacon_row(r, _):
                row = r * WINDOW
                for j in range(D // LANES):
                    sl = pl.ds(j * LANES, LANES)
                    rows[p, row, sl] = rows[p, row, sl] + b_v[sl]
                return 0

            lax.fori_loop(0, CHUNK // WINDOW, beacon_row, 0)

        for p in range(NBUF):
            gather_start(p, p)

        def outer(t, _):
            for p in range(NBUF):
                gather_wait(p)
                add_bias(p)
                store_start(t * NBUF + p, p)

            @pl.when(t != n_outer - 1)
            def _prefetch():
                for p in range(NBUF):
                    store_wait(p)
                    gather_start((t + 1) * NBUF + p, p)

            return 0

        lax.fori_loop(0, n_outer, outer, 0)
        for p in range(NBUF):
            store_wait(p)

    out = body(idx_flat, table, b_embed)
    return out.reshape(B, N, D)
